# baseline (speedup 1.0000x reference)
import jax
import jax.numpy as jnp
from jax import lax
from jax.experimental import pallas as pl
from jax.experimental.pallas import tpu as pltpu
from jax.experimental.pallas import tpu_sc as plsc

_NROWS = 4096 * 200
_TC_ROWS = 512000
_SC_ROWS = _NROWS - _TC_ROWS
_B = 4096
_NC, _NS = 2, 16
_NW = _NC * _NS
_RPW = _SC_ROWS // _NW
_R = 384
_CHUNKS = _RPW // _R
_NSTR = (_R * 5) // 128
_OPW = _RPW * 5 // 128


def _tc_body(in_ref, out_ref):
    i = lax.broadcasted_iota(jnp.int32, (_B, 5), 1)
    idx = jnp.where(
        i == 1, 5, jnp.where(i == 2, 17, jnp.where(i == 3, 42, jnp.where(i == 4, 99, 0)))
    )
    out_ref[...] = jnp.take_along_axis(in_ref[...], idx, axis=1)


def _sc_body(in_hbm, out_hbm, idxb, dstb, sem):
    wid = lax.axis_index("s") * _NC + lax.axis_index("c")
    base = wid * _RPW

    def pre(g, carry):
        j = lax.iota(jnp.int32, 16) + g * 16
        r = lax.shift_right_logical(j * 52429, 18)
        m = j - r * 5
        lane = jnp.where(
            m == 1, 5, jnp.where(m == 2, 17, jnp.where(m == 3, 42, jnp.where(m == 4, 99, 0)))
        )
        s = lax.shift_right_logical(g, 3)
        o = (g & 7) * 16
        idxb[s, pl.ds(o, 16)] = r * 128 + lane
        return carry

    lax.fori_loop(0, (_R * 5) // 16, pre, 0)

    def chunk(c, carry):
        r0 = base + c * _R
        src = in_hbm.at[pl.ds(r0 * 128, _R * 128)]
        handles = []
        for s in range(_NSTR):
            handles.append(pltpu.async_copy(src.at[idxb.at[s]], dstb.at[s], sem))
        for h in handles:
            h.wait()
        ob = wid * _OPW + c * _NSTR
        pltpu.sync_copy(dstb, out_hbm.at[pl.ds(ob, _NSTR)])
        return carry

    lax.fori_loop(0, _CHUNKS, chunk, 0)


@jax.jit
def kernel(inputs):
    x = inputs.reshape(_NROWS, 128)

    out_tc = pl.pallas_call(
        _tc_body,
        grid=(_TC_ROWS // _B,),
        in_specs=[pl.BlockSpec((_B, 128), lambda i: (i, 0))],
        out_specs=pl.BlockSpec((_B, 5), lambda i: (i, 0)),
        out_shape=jax.ShapeDtypeStruct((_TC_ROWS, 5), jnp.float32),
    )(x[:_TC_ROWS])

    mesh = plsc.VectorSubcoreMesh(
        core_axis_name="c", subcore_axis_name="s", num_cores=_NC, num_subcores=_NS
    )
    sc = pl.kernel(
        _sc_body,
        mesh=mesh,
        compiler_params=pltpu.CompilerParams(
            use_tc_tiling_on_sc=False, needs_layout_passes=False
        ),
        out_type=jax.ShapeDtypeStruct((_SC_ROWS * 5 // 128, 128), jnp.float32),
        scratch_types=[
            pltpu.VMEM((_NSTR, 128), jnp.int32),
            pltpu.VMEM((_NSTR, 128), jnp.float32),
            pltpu.SemaphoreType.DMA,
        ],
    )
    out_sc = sc(x[_TC_ROWS:].reshape(_SC_ROWS * 128))
    out = jnp.concatenate([out_tc, out_sc.reshape(_SC_ROWS, 5)], axis=0)
    return out.reshape(4096, 200, 5)


# speedup vs baseline: 1.2055x; 1.2055x over previous
import jax
import jax.numpy as jnp
from jax import lax
from jax.experimental import pallas as pl
from jax.experimental.pallas import tpu as pltpu
from jax.experimental.pallas import tpu_sc as plsc

_NROWS = 4096 * 200
_NC, _NS = 2, 16
_NW = _NC * _NS
_RPW = _NROWS // _NW
_R = 512
_CHUNKS = _RPW // _R
_NSTR = (_R * 5) // 128
_OPW = _RPW * 5 // 128


def _sc_body(in_hbm, out_hbm, idxb, d0, d1, s0, s1):
    wid = lax.axis_index("s") * _NC + lax.axis_index("c")
    base = wid * _RPW

    def pre(g, carry):
        j = lax.iota(jnp.int32, 16) + g * 16
        r = lax.shift_right_logical(j * 52429, 18)
        m = j - r * 5
        lane = jnp.where(
            m == 1, 5, jnp.where(m == 2, 17, jnp.where(m == 3, 42, jnp.where(m == 4, 99, 0)))
        )
        s = lax.shift_right_logical(g, 3)
        o = (g & 7) * 16
        idxb[s, pl.ds(o, 16)] = r * 128 + lane
        return carry

    lax.fori_loop(0, (_R * 5) // 16, pre, 0)

    def fire(c, buf, sem):
        src = in_hbm.at[pl.ds((base + c * _R) * 128, _R * 128)]
        for s in range(_NSTR):
            pltpu.async_copy(src.at[idxb.at[s]], buf.at[s], sem)

    def drain(c, buf, sem):
        src = in_hbm.at[pl.ds((base + c * _R) * 128, _R * 128)]
        for s in range(_NSTR):
            pltpu.make_async_copy(src.at[idxb.at[s]], buf.at[s], sem).wait()

    def store(c, buf):
        ob = wid * _OPW + c * _NSTR
        pltpu.sync_copy(buf, out_hbm.at[pl.ds(ob, _NSTR)])

    fire(0, d0, s0)
    fire(1, d1, s1)

    def pair(k, carry):
        c0 = 2 * k
        c1 = 2 * k + 1
        drain(c0, d0, s0)
        store(c0, d0)

        @pl.when(c0 + 2 < _CHUNKS)
        def _():
            fire(c0 + 2, d0, s0)

        drain(c1, d1, s1)
        store(c1, d1)

        @pl.when(c1 + 2 < _CHUNKS)
        def _():
            fire(c1 + 2, d1, s1)

        return carry

    lax.fori_loop(0, _CHUNKS // 2, pair, 0)


@jax.jit
def kernel(inputs):
    x = inputs.reshape(_NROWS * 128)
    mesh = plsc.VectorSubcoreMesh(
        core_axis_name="c", subcore_axis_name="s", num_cores=_NC, num_subcores=_NS
    )
    f = pl.kernel(
        _sc_body,
        mesh=mesh,
        compiler_params=pltpu.CompilerParams(
            use_tc_tiling_on_sc=False, needs_layout_passes=False
        ),
        out_type=jax.ShapeDtypeStruct((_NROWS * 5 // 128, 128), jnp.float32),
        scratch_types=[
            pltpu.VMEM((_NSTR, 128), jnp.int32),
            pltpu.VMEM((_NSTR, 128), jnp.float32),
            pltpu.VMEM((_NSTR, 128), jnp.float32),
            pltpu.SemaphoreType.DMA,
            pltpu.SemaphoreType.DMA,
        ],
    )
    out = f(x)
    return out.reshape(4096, 200, 5)
